# baseline (speedup 1.0000x reference)
import functools

import jax
import jax.numpy as jnp
from jax import lax
from jax.experimental import pallas as pl
from jax.experimental.pallas import tpu as pltpu, tpu_sc as plsc

N_ACTIONS = 16
Y_DIM = 32
BATCH = 16384

_info = plsc.get_sparse_core_info()
_NC, _NS = _info.num_cores, _info.num_subcores
_NW = _NC * _NS
_B_PER_W = BATCH // _NW

_mesh = plsc.VectorSubcoreMesh(core_axis_name="c", subcore_axis_name="s")


@functools.partial(
    pl.kernel,
    mesh=_mesh,
    compiler_params=pltpu.CompilerParams(use_tc_tiling_on_sc=False),
    out_type=(
        jax.ShapeDtypeStruct((BATCH, N_ACTIONS), jnp.float32),
        jax.ShapeDtypeStruct((BATCH, Y_DIM), jnp.float32),
    ),
    scratch_types=[
        pltpu.VMEM((_B_PER_W,), jnp.int32),
        pltpu.VMEM((_B_PER_W, N_ACTIONS), jnp.float32),
        pltpu.VMEM((_B_PER_W, Y_DIM), jnp.float32),
        pltpu.SemaphoreType.DMA,
        pltpu.SemaphoreType.DMA,
    ],
)
def _gather_two_tables(state_hbm, policy_hbm, y_hbm, pol_out, y_out,
                       idx_v, pol_v, y_v, sem_p, sem_y):
    wid = lax.axis_index("s") * _NC + lax.axis_index("c")
    base = wid * _B_PER_W
    pltpu.sync_copy(state_hbm.at[pl.ds(base, _B_PER_W)], idx_v)
    cp = pltpu.async_copy(policy_hbm.at[idx_v], pol_v, sem_p)
    cy = pltpu.async_copy(y_hbm.at[idx_v], y_v, sem_y)
    cp.wait()
    pltpu.sync_copy(pol_v, pol_out.at[pl.ds(base, _B_PER_W)])
    cy.wait()
    pltpu.sync_copy(y_v, y_out.at[pl.ds(base, _B_PER_W)])


def kernel(state, policy, y):
    return _gather_two_tables(state, policy, y)
